# baseline (speedup 1.0000x reference)
import functools

import jax
import jax.numpy as jnp
from jax import lax
from jax.experimental import pallas as pl
from jax.experimental.pallas import tpu as pltpu


def _fused_kernel(x2s_ref, x1s_ref, p1_ref, x1i_ref, p2_ref, W2_ref, W1_ref,
                  g2_ref, be2_ref, g1_ref, be1_ref, out_ref,
                  G2_ref, s2_ref, G1_ref, s1_ref, x2c_ref, h2s_ref,
                  sc1_ref, sh1_ref, *, nsteps, nb, M2, M1):
    f32 = jnp.float32
    g = pl.program_id(0)

    @pl.when(g == 0)
    def _init():
        G2_ref[...] = jnp.zeros_like(G2_ref)
        s2_ref[...] = jnp.zeros_like(s2_ref)
        G1_ref[...] = jnp.zeros_like(G1_ref)
        s1_ref[...] = jnp.zeros_like(s1_ref)

    @pl.when(g < nsteps)
    def _stats():
        x2 = x2s_ref[...]
        x1 = x1s_ref[...]
        G2_ref[...] += lax.dot_general(x2, x2, (((0,), (0,)), ((), ())),
                                       preferred_element_type=f32)
        s2_ref[...] += jnp.sum(x2, axis=0, keepdims=True)
        G1_ref[...] += lax.dot_general(x1, x1, (((0,), (0,)), ((), ())),
                                       preferred_element_type=f32)
        s1_ref[...] += jnp.sum(x1, axis=0, keepdims=True)
        c2 = x2.shape[0]
        x2c_ref[pl.ds(g * c2, c2), :] = x2

    @pl.when(g == nsteps - 1)
    def _fold_h2():
        def bn_fold(G, s, W, gg, be, M):
            C = G.shape[0]
            xbar = s * (1.0 / M)
            proj = lax.dot_general(xbar, W, (((1,), (1,)), ((), ())),
                                   preferred_element_type=f32)
            A = lax.dot_general(W, G, (((1,), (0,)), ((), ())),
                                preferred_element_type=f32)
            ones = jnp.ones((1, C), f32)
            ey2 = lax.dot_general(ones, A * W, (((1,), (1,)), ((), ())),
                                  preferred_element_type=f32) * (1.0 / M)
            var = ey2 - proj * proj
            scale = gg / jnp.sqrt(var + 1e-5)
            shift = be - proj * scale
            return scale, shift

        sc2, sh2 = bn_fold(G2_ref[...], s2_ref[...], W2_ref[...],
                           g2_ref[...], be2_ref[...], M2)
        sc1, sh1 = bn_fold(G1_ref[...], s1_ref[...], W1_ref[...],
                           g1_ref[...], be1_ref[...], M1)
        sc1_ref[...] = sc1
        sh1_ref[...] = sh1
        y2 = lax.dot_general(x2c_ref[...], W2_ref[...],
                             (((1,), (1,)), ((), ())),
                             preferred_element_type=f32)
        h2s_ref[...] = jnp.maximum(y2 * sc2 + sh2, 0.0)

    @pl.when(g >= nsteps)
    def _interp():
        b = (g - nsteps) // nb
        p1b = p1_ref[0]
        p2b = p2_ref[0]
        R = p1b.shape[0]
        N2 = p2b.shape[0]
        n1b = jnp.sum(p1b * p1b, axis=1, keepdims=True)
        n2c = jnp.sum(p2b * p2b, axis=1, keepdims=True)
        n2b = lax.transpose(n2c, (1, 0))

        inner2 = lax.dot_general(-2.0 * p1b, p2b,
                                 (((1,), (1,)), ((), ())),
                                 preferred_element_type=f32)
        d = inner2 + n2b

        wmat = jnp.zeros((R, N2), f32)
        rsum = jnp.zeros((R, 1), f32)
        for k in range(3):
            mk = jnp.min(d, axis=1, keepdims=True)
            sel = d == mk
            if k < 2:
                d = jnp.where(sel, jnp.inf, d)
            r = 1.0 / (jnp.maximum(mk + n1b, 0.0) + 1e-8)
            rsum = rsum + r
            wmat = wmat + jnp.where(sel, r, 0.0)

        h2b = h2s_ref[pl.ds(b * N2, N2), :]
        interp = lax.dot_general(wmat, h2b, (((1,), (0,)), ((), ())),
                                 preferred_element_type=f32)
        interp = interp * (1.0 / rsum)

        y1 = lax.dot_general(x1i_ref[0], W1_ref[...],
                             (((1,), (1,)), ((), ())),
                             preferred_element_type=f32)
        h1 = jnp.maximum(y1 * sc1_ref[...] + sh1_ref[...], 0.0)
        out_ref[0] = interp + h1


def kernel(p1, x1, p2, x2, W2, b2, g2, be2, W1, b1, g1, be1):
    B, N1, _ = p1.shape
    _, N2, C2 = x2.shape
    D = W2.shape[0]
    f32 = jnp.float32

    M2, M1 = B * N2, B * N1
    x2r = x2.reshape(M2, C2)
    x1r = x1.reshape(M1, D)
    g2r, be2r = g2.reshape(1, D), be2.reshape(1, D)
    g1r, be1r = g1.reshape(1, D), be1.reshape(1, D)

    SSTEPS = 8
    c2, c1 = M2 // SSTEPS, M1 // SSTEPS
    R = 512
    NB = N1 // R
    grid = (SSTEPS + B * NB,)
    S = SSTEPS

    def chunk_map(g):
        return (jnp.minimum(g, S - 1), 0)

    def blk_map(g):
        t = jnp.maximum(g - S, 0)
        return (t // NB, t % NB, 0)

    def b_map(g):
        t = jnp.maximum(g - S, 0)
        return (t // NB, 0, 0)

    def const2(g):
        return (0, 0)

    out = pl.pallas_call(
        functools.partial(_fused_kernel, nsteps=SSTEPS, nb=NB, M2=M2, M1=M1),
        grid=grid,
        in_specs=[
            pl.BlockSpec((c2, C2), chunk_map),
            pl.BlockSpec((c1, D), chunk_map),
            pl.BlockSpec((1, R, 3), blk_map),
            pl.BlockSpec((1, R, D), blk_map),
            pl.BlockSpec((1, N2, 3), b_map),
            pl.BlockSpec((D, C2), const2),
            pl.BlockSpec((D, D), const2),
            pl.BlockSpec((1, D), const2),
            pl.BlockSpec((1, D), const2),
            pl.BlockSpec((1, D), const2),
            pl.BlockSpec((1, D), const2),
        ],
        out_specs=pl.BlockSpec((1, R, D), blk_map),
        out_shape=jax.ShapeDtypeStruct((B, N1, D), f32),
        scratch_shapes=[
            pltpu.VMEM((C2, C2), f32), pltpu.VMEM((1, C2), f32),
            pltpu.VMEM((D, D), f32), pltpu.VMEM((1, D), f32),
            pltpu.VMEM((M2, C2), f32), pltpu.VMEM((M2, D), f32),
            pltpu.VMEM((1, D), f32), pltpu.VMEM((1, D), f32),
        ],
    )(x2r, x1r, p1, x1, p2, W2, W1, g2r, be2r, g1r, be1r)
    return out


# speedup vs baseline: 1.0369x; 1.0369x over previous
import functools

import jax
import jax.numpy as jnp
from jax import lax
from jax.experimental import pallas as pl
from jax.experimental.pallas import tpu as pltpu


def _fused_kernel(x2s_ref, x1s_ref, p1_ref, x1i_ref, p2_ref, W2_ref, W1_ref,
                  g2_ref, be2_ref, g1_ref, be1_ref, out_ref,
                  G2_ref, s2_ref, G1_ref, s1_ref, x2c_ref, h2s_ref,
                  sc1_ref, sh1_ref, *, nsteps, nb, M2, M1):
    f32 = jnp.float32
    g = pl.program_id(0)

    @pl.when(g == 0)
    def _init():
        G2_ref[...] = jnp.zeros_like(G2_ref)
        s2_ref[...] = jnp.zeros_like(s2_ref)
        G1_ref[...] = jnp.zeros_like(G1_ref)
        s1_ref[...] = jnp.zeros_like(s1_ref)

    @pl.when(g < nsteps)
    def _stats():
        x2 = x2s_ref[...]
        x1 = x1s_ref[...]
        G2_ref[...] += lax.dot_general(x2, x2, (((0,), (0,)), ((), ())),
                                       preferred_element_type=f32)
        s2_ref[...] += jnp.sum(x2, axis=0, keepdims=True)
        G1_ref[...] += lax.dot_general(x1, x1, (((0,), (0,)), ((), ())),
                                       preferred_element_type=f32)
        s1_ref[...] += jnp.sum(x1, axis=0, keepdims=True)
        c2 = x2.shape[0]
        x2c_ref[pl.ds(g * c2, c2), :] = x2

    @pl.when(g == nsteps - 1)
    def _fold_h2():
        def bn_fold(G, s, W, gg, be, M):
            C = G.shape[0]
            xbar = s * (1.0 / M)
            proj = lax.dot_general(xbar, W, (((1,), (1,)), ((), ())),
                                   preferred_element_type=f32)
            A = lax.dot_general(W, G, (((1,), (0,)), ((), ())),
                                preferred_element_type=f32)
            ones = jnp.ones((1, C), f32)
            ey2 = lax.dot_general(ones, A * W, (((1,), (1,)), ((), ())),
                                  preferred_element_type=f32) * (1.0 / M)
            var = ey2 - proj * proj
            scale = gg / jnp.sqrt(var + 1e-5)
            shift = be - proj * scale
            return scale, shift

        sc2, sh2 = bn_fold(G2_ref[...], s2_ref[...], W2_ref[...],
                           g2_ref[...], be2_ref[...], M2)
        sc1, sh1 = bn_fold(G1_ref[...], s1_ref[...], W1_ref[...],
                           g1_ref[...], be1_ref[...], M1)
        sc1_ref[...] = sc1
        sh1_ref[...] = sh1
        y2 = lax.dot_general(x2c_ref[...], W2_ref[...],
                             (((1,), (1,)), ((), ())),
                             preferred_element_type=f32)
        h2s_ref[...] = jnp.maximum(y2 * sc2 + sh2, 0.0)

    @pl.when(g >= nsteps)
    def _interp():
        b = (g - nsteps) // nb
        p1b = p1_ref[0]
        p2b = p2_ref[0]
        R = p1b.shape[0]
        N2 = p2b.shape[0]
        n1b = jnp.sum(p1b * p1b, axis=1, keepdims=True)
        n2c = jnp.sum(p2b * p2b, axis=1, keepdims=True)
        n2b = lax.transpose(n2c, (1, 0))

        inner2 = lax.dot_general(-2.0 * p1b, p2b,
                                 (((1,), (1,)), ((), ())),
                                 preferred_element_type=f32)
        d = inner2 + n2b

        wmat = jnp.zeros((R, N2), f32)
        rsum = jnp.zeros((R, 1), f32)
        for k in range(3):
            mk = jnp.min(d, axis=1, keepdims=True)
            sel = d == mk
            if k < 2:
                d = jnp.where(sel, jnp.inf, d)
            r = 1.0 / (jnp.maximum(mk + n1b, 0.0) + 1e-8)
            rsum = rsum + r
            wmat = wmat + jnp.where(sel, r, 0.0)

        h2b = h2s_ref[pl.ds(b * N2, N2), :]
        interp = lax.dot_general(wmat, h2b, (((1,), (0,)), ((), ())),
                                 preferred_element_type=f32)
        interp = interp * (1.0 / rsum)

        y1 = lax.dot_general(x1i_ref[0], W1_ref[...],
                             (((1,), (1,)), ((), ())),
                             preferred_element_type=f32)
        h1 = jnp.maximum(y1 * sc1_ref[...] + sh1_ref[...], 0.0)
        out_ref[0] = interp + h1


def kernel(p1, x1, p2, x2, W2, b2, g2, be2, W1, b1, g1, be1):
    B, N1, _ = p1.shape
    _, N2, C2 = x2.shape
    D = W2.shape[0]
    f32 = jnp.float32

    M2, M1 = B * N2, B * N1
    x2r = x2.reshape(M2, C2)
    x1r = x1.reshape(M1, D)
    g2r, be2r = g2.reshape(1, D), be2.reshape(1, D)
    g1r, be1r = g1.reshape(1, D), be1.reshape(1, D)

    SSTEPS = 8
    c2, c1 = M2 // SSTEPS, M1 // SSTEPS
    R = 1024
    NB = N1 // R
    grid = (SSTEPS + B * NB,)
    S = SSTEPS

    def chunk_map(g):
        return (jnp.minimum(g, S - 1), 0)

    def blk_map(g):
        t = jnp.maximum(g - S, 0)
        return (t // NB, t % NB, 0)

    def b_map(g):
        t = jnp.maximum(g - S, 0)
        return (t // NB, 0, 0)

    def const2(g):
        return (0, 0)

    out = pl.pallas_call(
        functools.partial(_fused_kernel, nsteps=SSTEPS, nb=NB, M2=M2, M1=M1),
        grid=grid,
        in_specs=[
            pl.BlockSpec((c2, C2), chunk_map),
            pl.BlockSpec((c1, D), chunk_map),
            pl.BlockSpec((1, R, 3), blk_map),
            pl.BlockSpec((1, R, D), blk_map),
            pl.BlockSpec((1, N2, 3), b_map),
            pl.BlockSpec((D, C2), const2),
            pl.BlockSpec((D, D), const2),
            pl.BlockSpec((1, D), const2),
            pl.BlockSpec((1, D), const2),
            pl.BlockSpec((1, D), const2),
            pl.BlockSpec((1, D), const2),
        ],
        out_specs=pl.BlockSpec((1, R, D), blk_map),
        out_shape=jax.ShapeDtypeStruct((B, N1, D), f32),
        scratch_shapes=[
            pltpu.VMEM((C2, C2), f32), pltpu.VMEM((1, C2), f32),
            pltpu.VMEM((D, D), f32), pltpu.VMEM((1, D), f32),
            pltpu.VMEM((M2, C2), f32), pltpu.VMEM((M2, D), f32),
            pltpu.VMEM((1, D), f32), pltpu.VMEM((1, D), f32),
        ],
    )(x2r, x1r, p1, x1, p2, W2, W1, g2r, be2r, g1r, be1r)
    return out
